# baseline (speedup 1.0000x reference)
import functools

import jax
import jax.numpy as jnp
from jax import lax
from jax.experimental import pallas as pl
from jax.experimental.pallas import tpu as pltpu
from jax.experimental.pallas import tpu_sc as plsc

NC = 2
NS = 16
NW = NC * NS
LANES = 16
EC = 128


@functools.lru_cache(maxsize=None)
def _make_sc_edge_agg(n_pad: int, d: int, er: int, rw0: int, n_tab: int = 0):
    n_tab = n_tab or n_pad
    rw1 = er // NS - rw0
    rwmax = max(rw0, rw1)
    rpt = n_pad // NS
    ZR = 16
    L = 2
    R = 2 * L
    assert rpt % ZR == 0 and er % NS == 0
    for rwc in (rw0, rw1):
        assert rwc >= 2 * L and rwc % 8 == 0

    mesh = plsc.VectorSubcoreMesh(
        core_axis_name="c", subcore_axis_name="s",
        num_cores=NC, num_subcores=NS)

    out_type = jax.ShapeDtypeStruct((NC, n_pad, d), jnp.float32)
    scratch = [
        pltpu.VMEM((rwmax, EC), jnp.int32),
        pltpu.VMEM((rwmax, EC), jnp.int32),
        pltpu.VMEM((R, EC, d), jnp.float32),
        pltpu.VMEM((ZR, d), jnp.float32),
        pltpu.VMEM_SHARED((n_pad, d), jnp.float32),
    ]
    scratch += [pltpu.SemaphoreType.DMA] * (2 * R)

    def body(z_hbm, src_hbm, dst_hbm, agg_out, *rest):
        sidx_v, didx_v, rows_v, zer_v, acc_sh = rest[:5]
        gsem = rest[5:5 + R]
        ssem = rest[5 + R:5 + 2 * R]
        c = lax.axis_index("c")
        s = lax.axis_index("s")

        def _fill(i, _):
            for k in range(d // LANES):
                zer_v[i, pl.ds(k * LANES, LANES)] = jnp.zeros((LANES,), jnp.float32)
            return _
        lax.fori_loop(0, ZR, _fill, 0)

        for r in range(rpt // ZR):
            pltpu.sync_copy(zer_v, acc_sh.at[pl.ds(s * rpt + r * ZR, ZR)])
        plsc.subcore_barrier()

        myrw = jnp.where(c == 0, rw0, rw1)
        base = jnp.where(c == 0, s * rw0, NS * rw0 + s * rw1)
        bs = jnp.minimum(base, er - rwmax)
        off = base - bs
        pltpu.sync_copy(src_hbm.at[pl.ds(bs, rwmax)], sidx_v)
        pltpu.sync_copy(dst_hbm.at[pl.ds(bs, rwmax)], didx_v)

        def gather(j, b):
            pltpu.async_copy(z_hbm.at[sidx_v.at[j + off]], rows_v.at[b],
                             gsem[b])

        def wait_gather(j, b):
            pltpu.make_async_copy(z_hbm.at[sidx_v.at[j + off]], rows_v.at[b],
                                  gsem[b]).wait()

        def scatter(j, b):
            pltpu.async_copy(rows_v.at[b], acc_sh.at[didx_v.at[j + off]],
                             ssem[b], add=True)

        def wait_scatter(j, b):
            pltpu.make_async_copy(rows_v.at[b], acc_sh.at[didx_v.at[j + off]],
                                  ssem[b]).wait()

        for j in range(L):
            gather(j, j)
        for j in range(L):
            wait_gather(j, j)
            scatter(j, j)
            gather(j + L, j + L)

        def _steady(jo, carry):
            for i in range(R):
                j = L + jo * R + i
                b = (L + i) % R
                wait_gather(j, b)
                scatter(j, b)
                bk = i
                wait_scatter(j - L, bk)
                gather(j + L, bk)
            return carry
        lax.fori_loop(0, (myrw - 2 * L) // R, _steady, 0)

        for t in range(L):
            j = myrw - L + t
            b = (L + t) % R
            wait_gather(j, b)
            scatter(j, b)
        for b in range(R):
            wait_scatter(myrw - R + b, b)
        plsc.subcore_barrier()

        pltpu.sync_copy(acc_sh.at[pl.ds(s * rpt, rpt)],
                        agg_out.at[c, pl.ds(s * rpt, rpt)])

    return pl.kernel(body, out_type=out_type, mesh=mesh,
                     scratch_types=scratch,
                     compiler_params=pltpu.CompilerParams(
                         use_tc_tiling_on_sc=False))


@functools.lru_cache(maxsize=None)
def _make_sc_deg(n_pad: int, er: int, rw0: int):
    rw1 = er // NS - rw0
    rwmax = max(rw0, rw1)
    rpt = n_pad // NS
    NSEM = 8
    assert rpt % EC == 0
    for rwc in (rw0, rw1):
        assert rwc % NSEM == 0 and rwc >= 2 * NSEM

    mesh = plsc.VectorSubcoreMesh(
        core_axis_name="c", subcore_axis_name="s",
        num_cores=NC, num_subcores=NS)

    out_type = jax.ShapeDtypeStruct((NC, n_pad, LANES), jnp.float32)
    scratch = [
        pltpu.VMEM((rwmax, EC), jnp.int32),
        pltpu.VMEM((EC, LANES), jnp.float32),
        pltpu.VMEM((EC, LANES), jnp.float32),
        pltpu.VMEM_SHARED((n_pad, LANES), jnp.float32),
    ]
    scratch += [pltpu.SemaphoreType.DMA] * NSEM

    def body(dst_hbm, deg_out, didx_v, ones_v, zer_v, deg_sh, *dsem):
        c = lax.axis_index("c")
        s = lax.axis_index("s")

        def _fill(i, _):
            ones_v[i, pl.ds(0, LANES)] = jnp.ones((LANES,), jnp.float32)
            zer_v[i, pl.ds(0, LANES)] = jnp.zeros((LANES,), jnp.float32)
            return _
        lax.fori_loop(0, EC, _fill, 0)
        for r in range(rpt // EC):
            pltpu.sync_copy(zer_v, deg_sh.at[pl.ds(s * rpt + r * EC, EC)])
        plsc.subcore_barrier()

        myrw = jnp.where(c == 0, rw0, rw1)
        base = jnp.where(c == 0, s * rw0, NS * rw0 + s * rw1)
        bs = jnp.minimum(base, er - rwmax)
        off = base - bs
        pltpu.sync_copy(dst_hbm.at[pl.ds(bs, rwmax)], didx_v)

        def scat(j, i):
            pltpu.async_copy(ones_v, deg_sh.at[didx_v.at[j + off]], dsem[i],
                             add=True)

        def wait_scat(j, i):
            pltpu.make_async_copy(ones_v, deg_sh.at[didx_v.at[j + off]],
                                  dsem[i]).wait()

        for i in range(NSEM):
            scat(i, i)

        def _steady(jo, carry):
            for i in range(NSEM):
                j = jo * NSEM + i
                wait_scat(j - NSEM, i)
                scat(j, i)
            return carry
        lax.fori_loop(1, myrw // NSEM, _steady, 0)
        for i in range(NSEM):
            wait_scat(myrw - NSEM + i, i)
        plsc.subcore_barrier()

        pltpu.sync_copy(deg_sh.at[pl.ds(s * rpt, rpt)],
                        deg_out.at[c, pl.ds(s * rpt, rpt)])

    return pl.kernel(body, out_type=out_type, mesh=mesh,
                     scratch_types=scratch,
                     compiler_params=pltpu.CompilerParams(
                         use_tc_tiling_on_sc=False))


_BLK = 1024


def _tc1_body(x_ref, w_ref, z_ref, r_ref, *, dh):
    acc = jnp.dot(x_ref[...], w_ref[...], preferred_element_type=jnp.float32)
    z_ref[...] = acc[:, :dh]
    r_ref[...] = acc[:, dh:]


def _tc2_body(aggp_ref, degp_ref, r1_ref, b1_ref, w2_ref, z2_ref, r2_ref, *, de):
    deg = jnp.maximum(degp_ref[0, :, 0:1] + degp_ref[1, :, 0:1], 1.0)
    agg = aggp_ref[0] + aggp_ref[1]
    h = jnp.maximum(agg / deg + r1_ref[...] + b1_ref[...], 0.0)
    acc = jnp.dot(h, w2_ref[...], preferred_element_type=jnp.float32)
    z2_ref[...] = acc[:, :de]
    r2_ref[...] = acc[:, de:]


def _tc3_body(aggp_ref, degp_ref, r2_ref, b2_ref, whp_ref, bh_ref,
              emb_ref, risk_ref):
    deg = jnp.maximum(degp_ref[0, :, 0:1] + degp_ref[1, :, 0:1], 1.0)
    emb = jnp.maximum((aggp_ref[0] + aggp_ref[1]) / deg + r2_ref[...]
                      + b2_ref[...], 0.0)
    emb_ref[...] = emb
    risk_ref[...] = jnp.dot(emb, whp_ref[...],
                            preferred_element_type=jnp.float32) + bh_ref[...]


def _row_spec(d):
    return pl.BlockSpec((_BLK, d), lambda i: (i, 0))


def _part_spec(d):
    return pl.BlockSpec((NC, _BLK, d), lambda i: (0, i, 0))


def _full_spec(*shape):
    return pl.BlockSpec(shape, lambda i: tuple(0 for _ in shape))


def kernel(x, edge_index, W1l, W1r, b1, W2l, W2r, b2, Wh, bh):
    n, d_in = x.shape
    dh = W1l.shape[1]
    de = W2l.shape[1]
    e = edge_index.shape[1]

    n_pad = (n // (NS * EC) + 1) * (NS * EC)
    rw = -(-e // (NW * EC * 8)) * 8
    er = rw * NW
    e_pad = er * EC

    src = edge_index[0]
    dst = edge_index[1]
    srcp = jnp.pad(src, (0, e_pad - e)).reshape(er, EC)
    dstp = jnp.pad(dst, (0, e_pad - e), constant_values=n).reshape(er, EC)
    x_p = jnp.pad(x, ((0, n_pad - n), (0, 0)))

    grid = (n_pad // _BLK,)

    w1 = jnp.concatenate([W1l, W1r], axis=1)
    z1, r1 = pl.pallas_call(
        functools.partial(_tc1_body, dh=dh),
        grid=grid,
        in_specs=[_row_spec(d_in), _full_spec(d_in, 2 * dh)],
        out_specs=[_row_spec(dh), _row_spec(dh)],
        out_shape=[jax.ShapeDtypeStruct((n_pad, dh), jnp.float32),
                   jax.ShapeDtypeStruct((n_pad, dh), jnp.float32)],
    )(x_p, w1)

    rw0 = 80
    K = 8
    offs = (jnp.arange(e_pad, dtype=jnp.int32) % K).reshape(er, EC) * n_pad
    srcp = srcp + offs
    degp = _make_sc_deg(n_pad, er, rw0)(dstp)
    z1r = jnp.concatenate([z1] * K, axis=0)
    agg1p = _make_sc_edge_agg(n_pad, dh, er, rw0, K * n_pad)(z1r, srcp, dstp)

    w2 = jnp.concatenate([W2l, W2r], axis=1)
    z2, r2 = pl.pallas_call(
        functools.partial(_tc2_body, de=de),
        grid=grid,
        in_specs=[_part_spec(dh), _part_spec(LANES), _row_spec(dh),
                  _full_spec(1, dh), _full_spec(dh, 2 * de)],
        out_specs=[_row_spec(de), _row_spec(de)],
        out_shape=[jax.ShapeDtypeStruct((n_pad, de), jnp.float32),
                   jax.ShapeDtypeStruct((n_pad, de), jnp.float32)],
    )(agg1p, degp, r1, b1.reshape(1, dh), w2)

    z2r = jnp.concatenate([z2] * K, axis=0)
    agg2p = _make_sc_edge_agg(n_pad, de, er, rw0, K * n_pad)(z2r, srcp, dstp)

    whp = jnp.pad(Wh, ((0, 0), (0, 128 - Wh.shape[1])))
    bhv = jnp.broadcast_to(bh, (1, 128)).astype(jnp.float32)
    emb, risk = pl.pallas_call(
        _tc3_body,
        grid=grid,
        in_specs=[_part_spec(de), _part_spec(LANES), _row_spec(de),
                  _full_spec(1, de), _full_spec(de, 128), _full_spec(1, 128)],
        out_specs=[_row_spec(de), _row_spec(128)],
        out_shape=[jax.ShapeDtypeStruct((n_pad, de), jnp.float32),
                   jax.ShapeDtypeStruct((n_pad, 128), jnp.float32)],
    )(agg2p, degp, r2, b2.reshape(1, de), whp, bhv)

    return (emb[:n], risk[:n, :1])


# speedup vs baseline: 1.0037x; 1.0037x over previous
import functools

import jax
import jax.numpy as jnp
from jax import lax
from jax.experimental import pallas as pl
from jax.experimental.pallas import tpu as pltpu
from jax.experimental.pallas import tpu_sc as plsc

NC = 2
NS = 16
NW = NC * NS
LANES = 16
EC = 128


@functools.lru_cache(maxsize=None)
def _make_sc_edge_agg(n_pad: int, d: int, er: int, rw0: int, n_tab: int = 0):
    n_tab = n_tab or n_pad
    rw1 = er // NS - rw0
    rwmax = max(rw0, rw1)
    rpt = n_pad // NS
    ZR = 16
    L = 2
    R = 2 * L
    assert rpt % ZR == 0 and er % NS == 0
    for rwc in (rw0, rw1):
        assert rwc >= 2 * L and rwc % 8 == 0

    mesh = plsc.VectorSubcoreMesh(
        core_axis_name="c", subcore_axis_name="s",
        num_cores=NC, num_subcores=NS)

    out_type = jax.ShapeDtypeStruct((NC, n_pad, d), jnp.float32)
    scratch = [
        pltpu.VMEM((rwmax, EC), jnp.int32),
        pltpu.VMEM((rwmax, EC), jnp.int32),
        pltpu.VMEM((R, EC, d), jnp.float32),
        pltpu.VMEM((ZR, d), jnp.float32),
        pltpu.VMEM_SHARED((n_pad, d), jnp.float32),
    ]
    scratch += [pltpu.SemaphoreType.DMA] * (2 * R)

    def body(z_hbm, src_hbm, dst_hbm, agg_out, *rest):
        sidx_v, didx_v, rows_v, zer_v, acc_sh = rest[:5]
        gsem = rest[5:5 + R]
        ssem = rest[5 + R:5 + 2 * R]
        c = lax.axis_index("c")
        s = lax.axis_index("s")

        def _fill(i, _):
            for k in range(d // LANES):
                zer_v[i, pl.ds(k * LANES, LANES)] = jnp.zeros((LANES,), jnp.float32)
            return _
        lax.fori_loop(0, ZR, _fill, 0)

        for r in range(rpt // ZR):
            pltpu.sync_copy(zer_v, acc_sh.at[pl.ds(s * rpt + r * ZR, ZR)])
        plsc.subcore_barrier()

        myrw = jnp.where(c == 0, rw0, rw1)
        base = jnp.where(c == 0, s * rw0, NS * rw0 + s * rw1)
        bs = jnp.minimum(base, er - rwmax)
        off = base - bs
        pltpu.sync_copy(src_hbm.at[pl.ds(bs, rwmax)], sidx_v)
        pltpu.sync_copy(dst_hbm.at[pl.ds(bs, rwmax)], didx_v)

        def gather(j, b):
            pltpu.async_copy(z_hbm.at[sidx_v.at[j + off]], rows_v.at[b],
                             gsem[b])

        def wait_gather(j, b):
            pltpu.make_async_copy(z_hbm.at[sidx_v.at[j + off]], rows_v.at[b],
                                  gsem[b]).wait()

        def scatter(j, b):
            pltpu.async_copy(rows_v.at[b], acc_sh.at[didx_v.at[j + off]],
                             ssem[b], add=True)

        def wait_scatter(j, b):
            pltpu.make_async_copy(rows_v.at[b], acc_sh.at[didx_v.at[j + off]],
                                  ssem[b]).wait()

        for j in range(L):
            gather(j, j)
        for j in range(L):
            wait_gather(j, j)
            scatter(j, j)
            gather(j + L, j + L)

        def _steady(jo, carry):
            for i in range(R):
                j = L + jo * R + i
                b = (L + i) % R
                wait_gather(j, b)
                scatter(j, b)
                bk = i
                wait_scatter(j - L, bk)
                gather(j + L, bk)
            return carry
        lax.fori_loop(0, (myrw - 2 * L) // R, _steady, 0)

        for t in range(L):
            j = myrw - L + t
            b = (L + t) % R
            wait_gather(j, b)
            scatter(j, b)
        for b in range(R):
            wait_scatter(myrw - R + b, b)
        plsc.subcore_barrier()

        pltpu.sync_copy(acc_sh.at[pl.ds(s * rpt, rpt)],
                        agg_out.at[c, pl.ds(s * rpt, rpt)])

    return pl.kernel(body, out_type=out_type, mesh=mesh,
                     scratch_types=scratch,
                     compiler_params=pltpu.CompilerParams(
                         use_tc_tiling_on_sc=False))


@functools.lru_cache(maxsize=None)
def _make_sc_deg(n_pad: int, er: int, rw0: int):
    rw1 = er // NS - rw0
    rwmax = max(rw0, rw1)
    rpt = n_pad // NS
    NSEM = 8
    assert rpt % EC == 0
    for rwc in (rw0, rw1):
        assert rwc % NSEM == 0 and rwc >= 2 * NSEM

    mesh = plsc.VectorSubcoreMesh(
        core_axis_name="c", subcore_axis_name="s",
        num_cores=NC, num_subcores=NS)

    out_type = jax.ShapeDtypeStruct((NC, n_pad, LANES), jnp.float32)
    scratch = [
        pltpu.VMEM((rwmax, EC), jnp.int32),
        pltpu.VMEM((EC, LANES), jnp.float32),
        pltpu.VMEM((EC, LANES), jnp.float32),
        pltpu.VMEM_SHARED((n_pad, LANES), jnp.float32),
    ]
    scratch += [pltpu.SemaphoreType.DMA] * NSEM

    def body(dst_hbm, deg_out, didx_v, ones_v, zer_v, deg_sh, *dsem):
        c = lax.axis_index("c")
        s = lax.axis_index("s")

        def _fill(i, _):
            ones_v[i, pl.ds(0, LANES)] = jnp.ones((LANES,), jnp.float32)
            zer_v[i, pl.ds(0, LANES)] = jnp.zeros((LANES,), jnp.float32)
            return _
        lax.fori_loop(0, EC, _fill, 0)
        for r in range(rpt // EC):
            pltpu.sync_copy(zer_v, deg_sh.at[pl.ds(s * rpt + r * EC, EC)])
        plsc.subcore_barrier()

        myrw = jnp.where(c == 0, rw0, rw1)
        base = jnp.where(c == 0, s * rw0, NS * rw0 + s * rw1)
        bs = jnp.minimum(base, er - rwmax)
        off = base - bs
        pltpu.sync_copy(dst_hbm.at[pl.ds(bs, rwmax)], didx_v)

        def scat(j, i):
            pltpu.async_copy(ones_v, deg_sh.at[didx_v.at[j + off]], dsem[i],
                             add=True)

        def wait_scat(j, i):
            pltpu.make_async_copy(ones_v, deg_sh.at[didx_v.at[j + off]],
                                  dsem[i]).wait()

        for i in range(NSEM):
            scat(i, i)

        def _steady(jo, carry):
            for i in range(NSEM):
                j = jo * NSEM + i
                wait_scat(j - NSEM, i)
                scat(j, i)
            return carry
        lax.fori_loop(1, myrw // NSEM, _steady, 0)
        for i in range(NSEM):
            wait_scat(myrw - NSEM + i, i)
        plsc.subcore_barrier()

        pltpu.sync_copy(deg_sh.at[pl.ds(s * rpt, rpt)],
                        deg_out.at[c, pl.ds(s * rpt, rpt)])

    return pl.kernel(body, out_type=out_type, mesh=mesh,
                     scratch_types=scratch,
                     compiler_params=pltpu.CompilerParams(
                         use_tc_tiling_on_sc=False))


_BLK = 1024


def _tc1_body(x_ref, w_ref, z_ref, r_ref, *, dh):
    acc = jnp.dot(x_ref[...], w_ref[...], preferred_element_type=jnp.float32)
    z_ref[...] = acc[:, :dh]
    r_ref[...] = acc[:, dh:]


def _tc2_body(aggp_ref, degp_ref, r1_ref, b1_ref, w2_ref, z2_ref, r2_ref, *, de):
    deg = jnp.maximum(degp_ref[0, :, 0:1] + degp_ref[1, :, 0:1], 1.0)
    agg = aggp_ref[0] + aggp_ref[1]
    h = jnp.maximum(agg / deg + r1_ref[...] + b1_ref[...], 0.0)
    acc = jnp.dot(h, w2_ref[...], preferred_element_type=jnp.float32)
    z2_ref[...] = acc[:, :de]
    r2_ref[...] = acc[:, de:]


def _tc3_body(aggp_ref, degp_ref, r2_ref, b2_ref, whp_ref, bh_ref,
              emb_ref, risk_ref):
    deg = jnp.maximum(degp_ref[0, :, 0:1] + degp_ref[1, :, 0:1], 1.0)
    emb = jnp.maximum((aggp_ref[0] + aggp_ref[1]) / deg + r2_ref[...]
                      + b2_ref[...], 0.0)
    emb_ref[...] = emb
    risk_ref[...] = jnp.dot(emb, whp_ref[...],
                            preferred_element_type=jnp.float32) + bh_ref[...]


def _row_spec(d):
    return pl.BlockSpec((_BLK, d), lambda i: (i, 0))


def _part_spec(d):
    return pl.BlockSpec((NC, _BLK, d), lambda i: (0, i, 0))


def _full_spec(*shape):
    return pl.BlockSpec(shape, lambda i: tuple(0 for _ in shape))


def kernel(x, edge_index, W1l, W1r, b1, W2l, W2r, b2, Wh, bh):
    n, d_in = x.shape
    dh = W1l.shape[1]
    de = W2l.shape[1]
    e = edge_index.shape[1]

    n_pad = (n // (NS * EC) + 1) * (NS * EC)
    rw = -(-e // (NW * EC * 8)) * 8
    er = rw * NW
    e_pad = er * EC

    src = edge_index[0]
    dst = edge_index[1]
    srcp = jnp.pad(src, (0, e_pad - e)).reshape(er, EC)
    dstp = jnp.pad(dst, (0, e_pad - e), constant_values=n).reshape(er, EC)
    x_p = jnp.pad(x, ((0, n_pad - n), (0, 0)))

    grid = (n_pad // _BLK,)

    w1 = jnp.concatenate([W1l, W1r], axis=1)
    z1, r1 = pl.pallas_call(
        functools.partial(_tc1_body, dh=dh),
        grid=grid,
        in_specs=[_row_spec(d_in), _full_spec(d_in, 2 * dh)],
        out_specs=[_row_spec(dh), _row_spec(dh)],
        out_shape=[jax.ShapeDtypeStruct((n_pad, dh), jnp.float32),
                   jax.ShapeDtypeStruct((n_pad, dh), jnp.float32)],
    )(x_p, w1)

    rw0 = 80
    K = 4
    offs = (jnp.arange(e_pad, dtype=jnp.int32) % K).reshape(er, EC) * n_pad
    srcp = srcp + offs
    degp = _make_sc_deg(n_pad, er, rw0)(dstp)
    z1r = jnp.concatenate([z1] * K, axis=0)
    agg1p = _make_sc_edge_agg(n_pad, dh, er, rw0, K * n_pad)(z1r, srcp, dstp)

    w2 = jnp.concatenate([W2l, W2r], axis=1)
    z2, r2 = pl.pallas_call(
        functools.partial(_tc2_body, de=de),
        grid=grid,
        in_specs=[_part_spec(dh), _part_spec(LANES), _row_spec(dh),
                  _full_spec(1, dh), _full_spec(dh, 2 * de)],
        out_specs=[_row_spec(de), _row_spec(de)],
        out_shape=[jax.ShapeDtypeStruct((n_pad, de), jnp.float32),
                   jax.ShapeDtypeStruct((n_pad, de), jnp.float32)],
    )(agg1p, degp, r1, b1.reshape(1, dh), w2)

    z2r = jnp.concatenate([z2] * K, axis=0)
    agg2p = _make_sc_edge_agg(n_pad, de, er, rw0, K * n_pad)(z2r, srcp, dstp)

    whp = jnp.pad(Wh, ((0, 0), (0, 128 - Wh.shape[1])))
    bhv = jnp.broadcast_to(bh, (1, 128)).astype(jnp.float32)
    emb, risk = pl.pallas_call(
        _tc3_body,
        grid=grid,
        in_specs=[_part_spec(de), _part_spec(LANES), _row_spec(de),
                  _full_spec(1, de), _full_spec(de, 128), _full_spec(1, 128)],
        out_specs=[_row_spec(de), _row_spec(128)],
        out_shape=[jax.ShapeDtypeStruct((n_pad, de), jnp.float32),
                   jax.ShapeDtypeStruct((n_pad, 128), jnp.float32)],
    )(agg2p, degp, r2, b2.reshape(1, de), whp, bhv)

    return (emb[:n], risk[:n, :1])


# speedup vs baseline: 1.0439x; 1.0401x over previous
import functools

import jax
import jax.numpy as jnp
from jax import lax
from jax.experimental import pallas as pl
from jax.experimental.pallas import tpu as pltpu
from jax.experimental.pallas import tpu_sc as plsc

NC = 2
NS = 16
NW = NC * NS
LANES = 16
EC = 128


@functools.lru_cache(maxsize=None)
def _make_sc_edge_agg(n_pad: int, d: int, er: int, rw0: int, n_tab: int = 0,
                      with_deg: bool = False):
    n_tab = n_tab or n_pad
    rw1 = er // NS - rw0
    rwmax = max(rw0, rw1)
    rpt = n_pad // NS
    ZR = 16
    L = 2
    R = 2 * L
    assert rpt % ZR == 0 and er % NS == 0
    for rwc in (rw0, rw1):
        assert rwc >= 2 * L and rwc % 8 == 0

    mesh = plsc.VectorSubcoreMesh(
        core_axis_name="c", subcore_axis_name="s",
        num_cores=NC, num_subcores=NS)

    out_type = [jax.ShapeDtypeStruct((NC, n_pad, d), jnp.float32)]
    scratch = [
        pltpu.VMEM((rwmax, EC), jnp.int32),
        pltpu.VMEM((rwmax, EC), jnp.int32),
        pltpu.VMEM((R, EC, d), jnp.float32),
        pltpu.VMEM((ZR, d), jnp.float32),
        pltpu.VMEM_SHARED((n_pad, d), jnp.float32),
    ]
    scratch += [pltpu.SemaphoreType.DMA] * (2 * R)
    if with_deg:
        out_type.append(jax.ShapeDtypeStruct((NC, n_pad, LANES), jnp.float32))
        scratch += [
            pltpu.VMEM((EC, LANES), jnp.float32),
            pltpu.VMEM((ZR, LANES), jnp.float32),
            pltpu.VMEM_SHARED((n_pad, LANES), jnp.float32),
        ]
        scratch += [pltpu.SemaphoreType.DMA] * R

    def body(z_hbm, src_hbm, dst_hbm, *rest):
        if with_deg:
            agg_out, deg_out = rest[0], rest[1]
            rest = rest[2:]
        else:
            agg_out = rest[0]
            rest = rest[1:]
        sidx_v, didx_v, rows_v, zer_v, acc_sh = rest[:5]
        gsem = rest[5:5 + R]
        ssem = rest[5 + R:5 + 2 * R]
        if with_deg:
            ones_v, zer16_v, deg_sh = rest[5 + 2 * R:8 + 2 * R]
            dsem = rest[8 + 2 * R:8 + 3 * R]
        c = lax.axis_index("c")
        s = lax.axis_index("s")

        def _fill(i, _):
            for k in range(d // LANES):
                zer_v[i, pl.ds(k * LANES, LANES)] = jnp.zeros((LANES,), jnp.float32)
            if with_deg:
                zer16_v[i, pl.ds(0, LANES)] = jnp.zeros((LANES,), jnp.float32)
            return _
        lax.fori_loop(0, ZR, _fill, 0)
        if with_deg:
            def _fillo(i, _):
                ones_v[i, pl.ds(0, LANES)] = jnp.ones((LANES,), jnp.float32)
                return _
            lax.fori_loop(0, EC, _fillo, 0)

        for r in range(rpt // ZR):
            pltpu.sync_copy(zer_v, acc_sh.at[pl.ds(s * rpt + r * ZR, ZR)])
            if with_deg:
                pltpu.sync_copy(zer16_v, deg_sh.at[pl.ds(s * rpt + r * ZR, ZR)])
        plsc.subcore_barrier()

        myrw = jnp.where(c == 0, rw0, rw1)
        base = jnp.where(c == 0, s * rw0, NS * rw0 + s * rw1)
        bs = jnp.minimum(base, er - rwmax)
        off = base - bs
        pltpu.sync_copy(src_hbm.at[pl.ds(bs, rwmax)], sidx_v)
        pltpu.sync_copy(dst_hbm.at[pl.ds(bs, rwmax)], didx_v)

        def gather(j, b):
            pltpu.async_copy(z_hbm.at[sidx_v.at[j + off]], rows_v.at[b],
                             gsem[b])

        def wait_gather(j, b):
            pltpu.make_async_copy(z_hbm.at[sidx_v.at[j + off]], rows_v.at[b],
                                  gsem[b]).wait()

        def scatter(j, b):
            pltpu.async_copy(rows_v.at[b], acc_sh.at[didx_v.at[j + off]],
                             ssem[b], add=True)
            if with_deg:
                pltpu.async_copy(ones_v, deg_sh.at[didx_v.at[j + off]],
                                 dsem[b], add=True)

        def wait_scatter(j, b):
            pltpu.make_async_copy(rows_v.at[b], acc_sh.at[didx_v.at[j + off]],
                                  ssem[b]).wait()
            if with_deg:
                pltpu.make_async_copy(ones_v, deg_sh.at[didx_v.at[j + off]],
                                      dsem[b]).wait()

        for j in range(L):
            gather(j, j)
        for j in range(L):
            wait_gather(j, j)
            scatter(j, j)
            gather(j + L, j + L)

        def _steady(jo, carry):
            for i in range(R):
                j = L + jo * R + i
                b = (L + i) % R
                wait_gather(j, b)
                scatter(j, b)
                bk = i
                wait_scatter(j - L, bk)
                gather(j + L, bk)
            return carry
        lax.fori_loop(0, (myrw - 2 * L) // R, _steady, 0)

        for t in range(L):
            j = myrw - L + t
            b = (L + t) % R
            wait_gather(j, b)
            scatter(j, b)
        for b in range(R):
            wait_scatter(myrw - R + b, b)
        plsc.subcore_barrier()

        pltpu.sync_copy(acc_sh.at[pl.ds(s * rpt, rpt)],
                        agg_out.at[c, pl.ds(s * rpt, rpt)])
        if with_deg:
            pltpu.sync_copy(deg_sh.at[pl.ds(s * rpt, rpt)],
                            deg_out.at[c, pl.ds(s * rpt, rpt)])

    return pl.kernel(body, out_type=tuple(out_type) if with_deg else out_type[0],
                     mesh=mesh,
                     scratch_types=scratch,
                     compiler_params=pltpu.CompilerParams(
                         use_tc_tiling_on_sc=False))


@functools.lru_cache(maxsize=None)
def _make_sc_deg(n_pad: int, er: int, rw0: int):
    rw1 = er // NS - rw0
    rwmax = max(rw0, rw1)
    rpt = n_pad // NS
    NSEM = 8
    assert rpt % EC == 0
    for rwc in (rw0, rw1):
        assert rwc % NSEM == 0 and rwc >= 2 * NSEM

    mesh = plsc.VectorSubcoreMesh(
        core_axis_name="c", subcore_axis_name="s",
        num_cores=NC, num_subcores=NS)

    out_type = jax.ShapeDtypeStruct((NC, n_pad, LANES), jnp.float32)
    scratch = [
        pltpu.VMEM((rwmax, EC), jnp.int32),
        pltpu.VMEM((EC, LANES), jnp.float32),
        pltpu.VMEM((EC, LANES), jnp.float32),
        pltpu.VMEM_SHARED((n_pad, LANES), jnp.float32),
    ]
    scratch += [pltpu.SemaphoreType.DMA] * NSEM

    def body(dst_hbm, deg_out, didx_v, ones_v, zer_v, deg_sh, *dsem):
        c = lax.axis_index("c")
        s = lax.axis_index("s")

        def _fill(i, _):
            ones_v[i, pl.ds(0, LANES)] = jnp.ones((LANES,), jnp.float32)
            zer_v[i, pl.ds(0, LANES)] = jnp.zeros((LANES,), jnp.float32)
            return _
        lax.fori_loop(0, EC, _fill, 0)
        for r in range(rpt // EC):
            pltpu.sync_copy(zer_v, deg_sh.at[pl.ds(s * rpt + r * EC, EC)])
        plsc.subcore_barrier()

        myrw = jnp.where(c == 0, rw0, rw1)
        base = jnp.where(c == 0, s * rw0, NS * rw0 + s * rw1)
        bs = jnp.minimum(base, er - rwmax)
        off = base - bs
        pltpu.sync_copy(dst_hbm.at[pl.ds(bs, rwmax)], didx_v)

        def scat(j, i):
            pltpu.async_copy(ones_v, deg_sh.at[didx_v.at[j + off]], dsem[i],
                             add=True)

        def wait_scat(j, i):
            pltpu.make_async_copy(ones_v, deg_sh.at[didx_v.at[j + off]],
                                  dsem[i]).wait()

        for i in range(NSEM):
            scat(i, i)

        def _steady(jo, carry):
            for i in range(NSEM):
                j = jo * NSEM + i
                wait_scat(j - NSEM, i)
                scat(j, i)
            return carry
        lax.fori_loop(1, myrw // NSEM, _steady, 0)
        for i in range(NSEM):
            wait_scat(myrw - NSEM + i, i)
        plsc.subcore_barrier()

        pltpu.sync_copy(deg_sh.at[pl.ds(s * rpt, rpt)],
                        deg_out.at[c, pl.ds(s * rpt, rpt)])

    return pl.kernel(body, out_type=out_type, mesh=mesh,
                     scratch_types=scratch,
                     compiler_params=pltpu.CompilerParams(
                         use_tc_tiling_on_sc=False))


_BLK = 1024


_REP = 4


def _tc1_body(x_ref, w_ref, z_ref, r_ref, *, dh):
    acc = jnp.dot(x_ref[...], w_ref[...], preferred_element_type=jnp.float32)
    for k in range(_REP):
        z_ref[k] = acc[:, :dh]
    r_ref[...] = acc[:, dh:]


def _tc2_body(aggp_ref, degp_ref, r1_ref, b1_ref, w2_ref, z2_ref, r2_ref, *, de):
    deg = jnp.maximum(degp_ref[0, :, 0:1] + degp_ref[1, :, 0:1], 1.0)
    agg = aggp_ref[0] + aggp_ref[1]
    h = jnp.maximum(agg / deg + r1_ref[...] + b1_ref[...], 0.0)
    acc = jnp.dot(h, w2_ref[...], preferred_element_type=jnp.float32)
    for k in range(_REP):
        z2_ref[k] = acc[:, :de]
    r2_ref[...] = acc[:, de:]


def _tc3_body(aggp_ref, degp_ref, r2_ref, b2_ref, whp_ref, bh_ref,
              emb_ref, risk_ref):
    deg = jnp.maximum(degp_ref[0, :, 0:1] + degp_ref[1, :, 0:1], 1.0)
    emb = jnp.maximum((aggp_ref[0] + aggp_ref[1]) / deg + r2_ref[...]
                      + b2_ref[...], 0.0)
    emb_ref[...] = emb
    risk_ref[...] = jnp.dot(emb, whp_ref[...],
                            preferred_element_type=jnp.float32) + bh_ref[...]


def _row_spec(d):
    return pl.BlockSpec((_BLK, d), lambda i: (i, 0))


def _part_spec(d):
    return pl.BlockSpec((NC, _BLK, d), lambda i: (0, i, 0))


def _full_spec(*shape):
    return pl.BlockSpec(shape, lambda i: tuple(0 for _ in shape))


def kernel(x, edge_index, W1l, W1r, b1, W2l, W2r, b2, Wh, bh):
    n, d_in = x.shape
    dh = W1l.shape[1]
    de = W2l.shape[1]
    e = edge_index.shape[1]

    n_pad = (n // (NS * EC) + 1) * (NS * EC)
    rw = -(-e // (NW * EC * 8)) * 8
    er = rw * NW
    e_pad = er * EC

    src = edge_index[0]
    dst = edge_index[1]
    srcp = jnp.pad(src, (0, e_pad - e)).reshape(er, EC)
    dstp = jnp.pad(dst, (0, e_pad - e), constant_values=n).reshape(er, EC)
    x_p = jnp.pad(x, ((0, n_pad - n), (0, 0)))

    grid = (n_pad // _BLK,)

    offs = (jnp.arange(e_pad, dtype=jnp.int32) % _REP).reshape(er, EC) * n_pad
    srcp = srcp + offs
    rw0 = 80

    w1 = jnp.concatenate([W1l, W1r], axis=1)
    z1r, r1 = pl.pallas_call(
        functools.partial(_tc1_body, dh=dh),
        grid=grid,
        in_specs=[_row_spec(d_in), _full_spec(d_in, 2 * dh)],
        out_specs=[pl.BlockSpec((_REP, _BLK, dh), lambda i: (0, i, 0)),
                   _row_spec(dh)],
        out_shape=[jax.ShapeDtypeStruct((_REP, n_pad, dh), jnp.float32),
                   jax.ShapeDtypeStruct((n_pad, dh), jnp.float32)],
    )(x_p, w1)

    agg1p, degp = _make_sc_edge_agg(n_pad, dh, er, rw0, _REP * n_pad, True)(
        z1r.reshape(_REP * n_pad, dh), srcp, dstp)

    w2 = jnp.concatenate([W2l, W2r], axis=1)
    z2r, r2 = pl.pallas_call(
        functools.partial(_tc2_body, de=de),
        grid=grid,
        in_specs=[_part_spec(dh), _part_spec(LANES), _row_spec(dh),
                  _full_spec(1, dh), _full_spec(dh, 2 * de)],
        out_specs=[pl.BlockSpec((_REP, _BLK, de), lambda i: (0, i, 0)),
                   _row_spec(de)],
        out_shape=[jax.ShapeDtypeStruct((_REP, n_pad, de), jnp.float32),
                   jax.ShapeDtypeStruct((n_pad, de), jnp.float32)],
    )(agg1p, degp, r1, b1.reshape(1, dh), w2)

    agg2p = _make_sc_edge_agg(n_pad, de, er, rw0, _REP * n_pad)(
        z2r.reshape(_REP * n_pad, de), srcp, dstp)

    whp = jnp.pad(Wh, ((0, 0), (0, 8 - Wh.shape[1])))
    bhv = jnp.broadcast_to(bh, (1, 8)).astype(jnp.float32)
    emb, risk = pl.pallas_call(
        _tc3_body,
        grid=grid,
        in_specs=[_part_spec(de), _part_spec(LANES), _row_spec(de),
                  _full_spec(1, de), _full_spec(de, 8), _full_spec(1, 8)],
        out_specs=[_row_spec(de), _row_spec(8)],
        out_shape=[jax.ShapeDtypeStruct((n_pad, de), jnp.float32),
                   jax.ShapeDtypeStruct((n_pad, 8), jnp.float32)],
    )(agg2p, degp, r2, b2.reshape(1, de), whp, bhv)

    return (emb[:n], risk[:n, :1])


# speedup vs baseline: 1.1157x; 1.0688x over previous
import functools

import jax
import jax.numpy as jnp
from jax import lax
from jax.experimental import pallas as pl
from jax.experimental.pallas import tpu as pltpu
from jax.experimental.pallas import tpu_sc as plsc

NC = 2
NS = 16
NW = NC * NS
LANES = 16
EC = 128


@functools.lru_cache(maxsize=None)
def _make_sc_edge_agg(n_pad: int, d: int, er: int, rw0: int, n_tab: int = 0,
                      with_deg: bool = False):
    n_tab = n_tab or n_pad
    rw1 = er // NS - rw0
    rwmax = max(rw0, rw1)
    rpt = n_pad // NS
    ZR = 16
    L = 2
    R = 2 * L
    assert rpt % ZR == 0 and er % NS == 0
    for rwc in (rw0, rw1):
        assert rwc >= 2 * L and rwc % 8 == 0

    mesh = plsc.VectorSubcoreMesh(
        core_axis_name="c", subcore_axis_name="s",
        num_cores=NC, num_subcores=NS)

    out_type = [jax.ShapeDtypeStruct((NC, n_pad, d), jnp.float32)]
    scratch = [
        pltpu.VMEM((rwmax, EC), jnp.int32),
        pltpu.VMEM((rwmax, EC), jnp.int32),
        pltpu.VMEM((R, EC, d), jnp.float32),
        pltpu.VMEM((ZR, d), jnp.float32),
        pltpu.VMEM_SHARED((n_pad, d), jnp.float32),
    ]
    scratch += [pltpu.SemaphoreType.DMA] * (2 * R)
    if with_deg:
        out_type.append(jax.ShapeDtypeStruct((NC, n_pad, LANES), jnp.float32))
        scratch += [
            pltpu.VMEM((EC, LANES), jnp.float32),
            pltpu.VMEM((ZR, LANES), jnp.float32),
            pltpu.VMEM_SHARED((n_pad, LANES), jnp.float32),
        ]
        scratch += [pltpu.SemaphoreType.DMA] * R

    def body(z_hbm, src_hbm, dst_hbm, *rest):
        if with_deg:
            agg_out, deg_out = rest[0], rest[1]
            rest = rest[2:]
        else:
            agg_out = rest[0]
            rest = rest[1:]
        sidx_v, didx_v, rows_v, zer_v, acc_sh = rest[:5]
        gsem = rest[5:5 + R]
        ssem = rest[5 + R:5 + 2 * R]
        if with_deg:
            ones_v, zer16_v, deg_sh = rest[5 + 2 * R:8 + 2 * R]
            dsem = rest[8 + 2 * R:8 + 3 * R]
        c = lax.axis_index("c")
        s = lax.axis_index("s")

        def _fill(i, _):
            for k in range(d // LANES):
                zer_v[i, pl.ds(k * LANES, LANES)] = jnp.zeros((LANES,), jnp.float32)
            if with_deg:
                zer16_v[i, pl.ds(0, LANES)] = jnp.zeros((LANES,), jnp.float32)
            return _
        lax.fori_loop(0, ZR, _fill, 0)
        if with_deg:
            def _fillo(i, _):
                ones_v[i, pl.ds(0, LANES)] = jnp.ones((LANES,), jnp.float32)
                return _
            lax.fori_loop(0, EC, _fillo, 0)

        for r in range(rpt // ZR):
            pltpu.sync_copy(zer_v, acc_sh.at[pl.ds(s * rpt + r * ZR, ZR)])
            if with_deg:
                pltpu.sync_copy(zer16_v, deg_sh.at[pl.ds(s * rpt + r * ZR, ZR)])
        plsc.subcore_barrier()

        myrw = jnp.where(c == 0, rw0, rw1)
        base = jnp.where(c == 0, s * rw0, NS * rw0 + s * rw1)
        bs = jnp.minimum(base, er - rwmax)
        off = base - bs
        pltpu.sync_copy(src_hbm.at[pl.ds(bs, rwmax)], sidx_v)
        pltpu.sync_copy(dst_hbm.at[pl.ds(bs, rwmax)], didx_v)

        def gather(j, b):
            pltpu.async_copy(z_hbm.at[sidx_v.at[j + off]], rows_v.at[b],
                             gsem[b])

        def wait_gather(j, b):
            pltpu.make_async_copy(z_hbm.at[sidx_v.at[j + off]], rows_v.at[b],
                                  gsem[b]).wait()

        def scatter(j, b):
            pltpu.async_copy(rows_v.at[b], acc_sh.at[didx_v.at[j + off]],
                             ssem[b], add=True)
            if with_deg:
                pltpu.async_copy(ones_v, deg_sh.at[didx_v.at[j + off]],
                                 dsem[b], add=True)

        def wait_scatter(j, b):
            pltpu.make_async_copy(rows_v.at[b], acc_sh.at[didx_v.at[j + off]],
                                  ssem[b]).wait()
            if with_deg:
                pltpu.make_async_copy(ones_v, deg_sh.at[didx_v.at[j + off]],
                                      dsem[b]).wait()

        for j in range(L):
            gather(j, j)
        for j in range(L):
            wait_gather(j, j)
            scatter(j, j)
            gather(j + L, j + L)

        def _steady(jo, carry):
            for i in range(R):
                j = L + jo * R + i
                b = (L + i) % R
                wait_gather(j, b)
                scatter(j, b)
                bk = i
                wait_scatter(j - L, bk)
                gather(j + L, bk)
            return carry
        lax.fori_loop(0, (myrw - 2 * L) // R, _steady, 0)

        for t in range(L):
            j = myrw - L + t
            b = (L + t) % R
            wait_gather(j, b)
            scatter(j, b)
        for b in range(R):
            wait_scatter(myrw - R + b, b)
        plsc.subcore_barrier()

        pltpu.sync_copy(acc_sh.at[pl.ds(s * rpt, rpt)],
                        agg_out.at[c, pl.ds(s * rpt, rpt)])
        if with_deg:
            pltpu.sync_copy(deg_sh.at[pl.ds(s * rpt, rpt)],
                            deg_out.at[c, pl.ds(s * rpt, rpt)])

    return pl.kernel(body, out_type=tuple(out_type) if with_deg else out_type[0],
                     mesh=mesh,
                     scratch_types=scratch,
                     compiler_params=pltpu.CompilerParams(
                         use_tc_tiling_on_sc=False))


@functools.lru_cache(maxsize=None)
def _make_sc_deg(n_pad: int, er: int, rw0: int):
    rw1 = er // NS - rw0
    rwmax = max(rw0, rw1)
    rpt = n_pad // NS
    NSEM = 8
    assert rpt % EC == 0
    for rwc in (rw0, rw1):
        assert rwc % NSEM == 0 and rwc >= 2 * NSEM

    mesh = plsc.VectorSubcoreMesh(
        core_axis_name="c", subcore_axis_name="s",
        num_cores=NC, num_subcores=NS)

    out_type = jax.ShapeDtypeStruct((NC, n_pad, LANES), jnp.float32)
    scratch = [
        pltpu.VMEM((rwmax, EC), jnp.int32),
        pltpu.VMEM((EC, LANES), jnp.float32),
        pltpu.VMEM((EC, LANES), jnp.float32),
        pltpu.VMEM_SHARED((n_pad, LANES), jnp.float32),
    ]
    scratch += [pltpu.SemaphoreType.DMA] * NSEM

    def body(dst_hbm, deg_out, didx_v, ones_v, zer_v, deg_sh, *dsem):
        c = lax.axis_index("c")
        s = lax.axis_index("s")

        def _fill(i, _):
            ones_v[i, pl.ds(0, LANES)] = jnp.ones((LANES,), jnp.float32)
            zer_v[i, pl.ds(0, LANES)] = jnp.zeros((LANES,), jnp.float32)
            return _
        lax.fori_loop(0, EC, _fill, 0)
        for r in range(rpt // EC):
            pltpu.sync_copy(zer_v, deg_sh.at[pl.ds(s * rpt + r * EC, EC)])
        plsc.subcore_barrier()

        myrw = jnp.where(c == 0, rw0, rw1)
        base = jnp.where(c == 0, s * rw0, NS * rw0 + s * rw1)
        bs = jnp.minimum(base, er - rwmax)
        off = base - bs
        pltpu.sync_copy(dst_hbm.at[pl.ds(bs, rwmax)], didx_v)

        def scat(j, i):
            pltpu.async_copy(ones_v, deg_sh.at[didx_v.at[j + off]], dsem[i],
                             add=True)

        def wait_scat(j, i):
            pltpu.make_async_copy(ones_v, deg_sh.at[didx_v.at[j + off]],
                                  dsem[i]).wait()

        for i in range(NSEM):
            scat(i, i)

        def _steady(jo, carry):
            for i in range(NSEM):
                j = jo * NSEM + i
                wait_scat(j - NSEM, i)
                scat(j, i)
            return carry
        lax.fori_loop(1, myrw // NSEM, _steady, 0)
        for i in range(NSEM):
            wait_scat(myrw - NSEM + i, i)
        plsc.subcore_barrier()

        pltpu.sync_copy(deg_sh.at[pl.ds(s * rpt, rpt)],
                        deg_out.at[c, pl.ds(s * rpt, rpt)])

    return pl.kernel(body, out_type=out_type, mesh=mesh,
                     scratch_types=scratch,
                     compiler_params=pltpu.CompilerParams(
                         use_tc_tiling_on_sc=False))


_BLK = 1024


_REP = 4


def _tc1_body(x_ref, w_ref, z_ref, r_ref, *, dh):
    acc = jnp.dot(x_ref[...], w_ref[...], preferred_element_type=jnp.float32)
    for k in range(_REP):
        z_ref[k] = acc[:, :dh]
    r_ref[...] = acc[:, dh:]


def _tc2_body(aggp_ref, degp_ref, r1_ref, b1_ref, w2_ref, z2_ref, r2_ref, *, de):
    deg = jnp.maximum(degp_ref[0, :, 0:1] + degp_ref[1, :, 0:1], 1.0)
    agg = aggp_ref[0] + aggp_ref[1]
    h = jnp.maximum(agg / deg + r1_ref[...] + b1_ref[...], 0.0)
    acc = jnp.dot(h, w2_ref[...], preferred_element_type=jnp.float32)
    for k in range(_REP):
        z2_ref[k] = acc[:, :de]
    r2_ref[...] = acc[:, de:]


def _tc3_body(aggp_ref, degp_ref, r2_ref, b2_ref, whp_ref, bh_ref,
              emb_ref, risk_ref):
    deg = jnp.maximum(degp_ref[0, :, 0:1] + degp_ref[1, :, 0:1], 1.0)
    emb = jnp.maximum((aggp_ref[0] + aggp_ref[1]) / deg + r2_ref[...]
                      + b2_ref[...], 0.0)
    emb_ref[...] = emb
    risk_ref[...] = jnp.dot(emb, whp_ref[...],
                            preferred_element_type=jnp.float32) + bh_ref[...]


def _row_spec(d):
    return pl.BlockSpec((_BLK, d), lambda i: (i, 0))


def _part_spec(d):
    return pl.BlockSpec((NC, _BLK, d), lambda i: (0, i, 0))


def _full_spec(*shape):
    return pl.BlockSpec(shape, lambda i: tuple(0 for _ in shape))


def kernel(x, edge_index, W1l, W1r, b1, W2l, W2r, b2, Wh, bh):
    n, d_in = x.shape
    dh = W1l.shape[1]
    de = W2l.shape[1]
    e = edge_index.shape[1]

    n_pad = (n // (NS * EC) + 1) * (NS * EC)
    rw = -(-e // (NW * EC * 8)) * 8
    er = rw * NW
    e_pad = er * EC

    src = edge_index[0]
    dst = edge_index[1]
    srcp = jnp.pad(src, (0, e_pad - e)).reshape(er, EC)
    dstp = jnp.pad(dst, (0, e_pad - e), constant_values=n).reshape(er, EC)
    x_p = jnp.pad(x, ((0, n_pad - n), (0, 0)))

    grid = (n_pad // _BLK,)

    offs = (jnp.arange(e_pad, dtype=jnp.int32) % _REP).reshape(er, EC) * n_pad
    srcp = srcp + offs
    rw0 = 104

    w1 = jnp.concatenate([W1l, W1r], axis=1)
    z1r, r1 = pl.pallas_call(
        functools.partial(_tc1_body, dh=dh),
        grid=grid,
        in_specs=[_row_spec(d_in), _full_spec(d_in, 2 * dh)],
        out_specs=[pl.BlockSpec((_REP, _BLK, dh), lambda i: (0, i, 0)),
                   _row_spec(dh)],
        out_shape=[jax.ShapeDtypeStruct((_REP, n_pad, dh), jnp.float32),
                   jax.ShapeDtypeStruct((n_pad, dh), jnp.float32)],
    )(x_p, w1)

    agg1p, degp = _make_sc_edge_agg(n_pad, dh, er, rw0, _REP * n_pad, True)(
        z1r.reshape(_REP * n_pad, dh), srcp, dstp)

    w2 = jnp.concatenate([W2l, W2r], axis=1)
    z2r, r2 = pl.pallas_call(
        functools.partial(_tc2_body, de=de),
        grid=grid,
        in_specs=[_part_spec(dh), _part_spec(LANES), _row_spec(dh),
                  _full_spec(1, dh), _full_spec(dh, 2 * de)],
        out_specs=[pl.BlockSpec((_REP, _BLK, de), lambda i: (0, i, 0)),
                   _row_spec(de)],
        out_shape=[jax.ShapeDtypeStruct((_REP, n_pad, de), jnp.float32),
                   jax.ShapeDtypeStruct((n_pad, de), jnp.float32)],
    )(agg1p, degp, r1, b1.reshape(1, dh), w2)

    agg2p = _make_sc_edge_agg(n_pad, de, er, rw0, _REP * n_pad)(
        z2r.reshape(_REP * n_pad, de), srcp, dstp)

    whp = jnp.pad(Wh, ((0, 0), (0, 8 - Wh.shape[1])))
    bhv = jnp.broadcast_to(bh, (1, 8)).astype(jnp.float32)
    emb, risk = pl.pallas_call(
        _tc3_body,
        grid=grid,
        in_specs=[_part_spec(de), _part_spec(LANES), _row_spec(de),
                  _full_spec(1, de), _full_spec(de, 8), _full_spec(1, 8)],
        out_specs=[_row_spec(de), _row_spec(8)],
        out_shape=[jax.ShapeDtypeStruct((n_pad, de), jnp.float32),
                   jax.ShapeDtypeStruct((n_pad, 8), jnp.float32)],
    )(agg2p, degp, r2, b2.reshape(1, de), whp, bhv)

    return (emb[:n], risk[:n, :1])
